# baseline (speedup 1.0000x reference)
import functools

import jax
import jax.numpy as jnp
from jax import lax
from jax.experimental import pallas as pl
from jax.experimental.pallas import tpu as pltpu
from jax.experimental.pallas import tpu_sc as plsc

_B = 1024
_D = 32
_VT = 2048
_C = 512
_NW = 32
_SW = _B // _NW


def _q2_body(x_ref, w2_ref, b2_ref, o_ref, oxc_ref):
    xb = x_ref[...]
    q2 = lax.dot_general(w2_ref[...], xb, (((1,), (1,)), ((), ())),
                         preferred_element_type=jnp.float32) + b2_ref[...]
    o_ref[...] = q2[None]
    oxc_ref[...] = xb[None]


def _sc_prep_body(xc_hbm, q2c_hbm, batch_hbm, wts_hbm, sv_hbm, out_hbm,
                  batch_v, wts_v, sv_v, vn_v, q1s_v, sg_v,
                  x_v, q2_v, alpha_v, sh_v):
    n = batch_hbm.shape[0]
    wid = lax.axis_index("s") * 2 + lax.axis_index("c")
    s0 = wid * _SW

    pltpu.sync_copy(batch_hbm, batch_v)
    pltpu.sync_copy(wts_hbm, wts_v)
    pltpu.sync_copy(sv_hbm, sv_v)

    iota16 = lax.iota(jnp.int32, 16)

    def lower_bound(thr):
        pos = jnp.zeros((16,), jnp.int32)
        for j in range(14, -1, -1):
            npos = pos + jnp.int32(1 << j)
            idx = jnp.minimum(npos - 1, n - 1)
            val = plsc.load_gather(batch_v, [idx])
            pred = (npos <= n) & (val < thr)
            pos = jnp.where(pred, npos, pos)
        return pos

    e0 = lower_bound(s0 + 1 + iota16)
    e1 = lower_bound(s0 + 17 + iota16)
    lo = lower_bound(jnp.full((16,), s0, jnp.int32))
    t_lo = lo[0]
    t_hi = e1[15]

    tv0 = jnp.where(e0 - 1 < 0, n - 1, e0 - 1)
    tv1 = jnp.where(e1 - 1 < 0, n - 1, e1 - 1)

    c_lo = t_lo // _C
    c_hi = (t_hi + _C - 1) // _C
    tmin = jnp.minimum(jnp.min(tv0), jnp.min(tv1))
    tmax = jnp.maximum(jnp.max(tv0), jnp.max(tv1))
    a_lo = jnp.minimum(c_lo, tmin // _C)
    a_hi = jnp.maximum(c_hi, tmax // _C + 1)

    def chunk_vn(c, carry):
        base = c * _C
        pltpu.sync_copy(xc_hbm.at[c], x_v)
        for sloc in range(_SW):
            t = tv0[sloc] if sloc < 16 else tv1[sloc - 16]

            @pl.when((t >= base) & (t < base + _C))
            def _copy_row(sloc=sloc, t=t, base=base):
                vn_v[pl.ds(sloc * _D, 16)] = x_v[t - base, pl.ds(0, 16)]
                vn_v[pl.ds(sloc * _D + 16, 16)] = x_v[t - base, pl.ds(16, 16)]

        return carry

    lax.fori_loop(a_lo, a_hi, chunk_vn, 0)

    zero16 = jnp.zeros((16,), jnp.int32)

    def q1s_one(sloc, carry):
        acc0 = sv_v[pl.ds(0, 16)]
        acc1 = sv_v[pl.ds(16, 16)]
        for k in range(_D):
            sck = plsc.load_gather(vn_v, [zero16 + (sloc * _D + k)])
            acc0 = acc0 + sck * wts_v[pl.ds(k * _D, 16)]
            acc1 = acc1 + sck * wts_v[pl.ds(k * _D + 16, 16)]
        q1s_v[pl.ds(sloc * _D, 16)] = acc0
        q1s_v[pl.ds(sloc * _D + 16, 16)] = acc1
        return carry

    lax.fori_loop(0, _SW, q1s_one, 0)

    def zero_one(i, carry):
        sg_v[pl.ds(i * 16, 16)] = jnp.zeros((16,), jnp.float32)
        return carry

    lax.fori_loop(0, _SW * _D // 16, zero_one, 0)

    def chunk(c, carry):
        base = c * _C
        pltpu.sync_copy(xc_hbm.at[c], x_v)
        pltpu.sync_copy(q2c_hbm.at[c], q2_v)
        j0 = jnp.maximum(t_lo - base, 0)
        j1 = jnp.minimum(t_hi - base, _C)

        def blk(g, carry2):
            off = g * 16
            tok = off + iota16
            msk = (tok >= j0) & (tok < j1)
            segv = batch_v[pl.ds(base + off, 16)]
            sloc = jnp.clip(segv - s0, 0, _SW - 1)
            acc = sv_v[pl.ds(96, 16)]
            qw0 = sv_v[pl.ds(32, 16)]
            qw1 = sv_v[pl.ds(48, 16)]
            for dd in range(_D):
                q1g = plsc.load_gather(q1s_v, [sloc * _D + dd])
                sgm = 1.0 / (1.0 + jnp.exp(-(q1g + q2_v[dd, pl.ds(off, 16)])))
                qwd = qw0[dd] if dd < 16 else qw1[dd - 16]
                acc = acc + qwd * sgm
            alpha_v[pl.ds(0, 16)] = jnp.where(msk, acc, 0.0)
            for j in range(16):
                a = plsc.load_gather(alpha_v, [zero16 + j])
                slj = sloc[j]
                r0 = sg_v[pl.ds(slj * _D, 16)]
                r1 = sg_v[pl.ds(slj * _D + 16, 16)]
                sg_v[pl.ds(slj * _D, 16)] = r0 + a * x_v[off + j, pl.ds(0, 16)]
                sg_v[pl.ds(slj * _D + 16, 16)] = (
                    r1 + a * x_v[off + j, pl.ds(16, 16)])
            return carry2

        lax.fori_loop(0, _C // 16, blk, 0)
        return carry

    lax.fori_loop(c_lo, c_hi, chunk, 0)

    def sh_one(sloc, carry):
        acc0 = sv_v[pl.ds(64, 16)]
        acc1 = sv_v[pl.ds(80, 16)]
        for k in range(_D):
            vv = plsc.load_gather(vn_v, [zero16 + (sloc * _D + k)])
            gg = plsc.load_gather(sg_v, [zero16 + (sloc * _D + k)])
            acc0 = (acc0 + vv * wts_v[pl.ds(1024 + k * _D, 16)]
                    + gg * wts_v[pl.ds(2048 + k * _D, 16)])
            acc1 = (acc1 + vv * wts_v[pl.ds(1024 + k * _D + 16, 16)]
                    + gg * wts_v[pl.ds(2048 + k * _D + 16, 16)])
        sh_v[sloc, pl.ds(0, 16)] = acc0
        sh_v[sloc, pl.ds(16, 16)] = acc1
        return carry

    lax.fori_loop(0, _SW, sh_one, 0)
    pltpu.sync_copy(sh_v, out_hbm.at[pl.ds(s0, _SW)])


_sc_prep = functools.partial(
    pl.kernel,
    _sc_prep_body,
    out_type=jax.ShapeDtypeStruct((_B, _D), jnp.float32),
    mesh=plsc.VectorSubcoreMesh(core_axis_name="c", subcore_axis_name="s"),
    compiler_params=pltpu.CompilerParams(needs_layout_passes=False),
    scratch_types=[
        pltpu.VMEM((16384,), jnp.int32),
        pltpu.VMEM((3 * _D * _D,), jnp.float32),
        pltpu.VMEM((128,), jnp.float32),
        pltpu.VMEM((_SW * _D,), jnp.float32),
        pltpu.VMEM((_SW * _D,), jnp.float32),
        pltpu.VMEM((_SW * _D,), jnp.float32),
        pltpu.VMEM((_C, _D), jnp.float32),
        pltpu.VMEM((_D, _C), jnp.float32),
        pltpu.VMEM((16,), jnp.float32),
        pltpu.VMEM((_SW, _D), jnp.float32),
    ],
)


def _score_body(e_ref, sh_ref, out_ref):
    out_ref[...] = lax.dot_general(e_ref[...].astype(jnp.bfloat16),
                                   sh_ref[...].astype(jnp.bfloat16),
                                   (((1,), (1,)), ((), ())),
                                   preferred_element_type=jnp.float32)


def kernel(session_embedding, all_item_embedding, batch,
           W1_w, W1_b, W2_w, W2_b, q_w, q_b, W3_w, W3_b):
    n, d = session_embedding.shape
    v = all_item_embedding.shape[0]

    batch = batch.astype(jnp.int32)

    q2c, xc = pl.pallas_call(
        _q2_body,
        grid=(n // _C,),
        in_specs=[pl.BlockSpec((_C, d), lambda c: (c, 0)),
                  pl.BlockSpec((d, d), lambda c: (0, 0)),
                  pl.BlockSpec((d, 1), lambda c: (0, 0))],
        out_specs=[pl.BlockSpec((1, d, _C), lambda c: (c, 0, 0)),
                   pl.BlockSpec((1, _C, d), lambda c: (c, 0, 0))],
        out_shape=[jax.ShapeDtypeStruct((n // _C, d, _C), jnp.float32),
                   jax.ShapeDtypeStruct((n // _C, _C, d), jnp.float32)],
    )(session_embedding, W2_w, W2_b[:, None])

    wts = jnp.concatenate([W1_w.T, W3_w[:, :d].T, W3_w[:, d:].T],
                          axis=0).reshape(-1)
    sv = jnp.concatenate([W1_b, q_w[0], W3_b,
                          jnp.full((32,), q_b[0], jnp.float32)])

    sh = _sc_prep()(xc, q2c, batch, wts, sv)

    nvt = pl.cdiv(v, _VT)
    zt = pl.pallas_call(
        _score_body,
        grid=(nvt,),
        in_specs=[pl.BlockSpec((_VT, d), lambda i: (i, 0)),
                  pl.BlockSpec((_B, d), lambda i: (0, 0))],
        out_specs=pl.BlockSpec((_VT, _B), lambda i: (i, 0)),
        out_shape=jax.ShapeDtypeStruct((v, _B), jnp.float32),
        compiler_params=pltpu.CompilerParams(
            dimension_semantics=("arbitrary",)),
    )(all_item_embedding, sh)
    return zt.T


# speedup vs baseline: 1.0093x; 1.0093x over previous
import functools

import jax
import jax.numpy as jnp
from jax import lax
from jax.experimental import pallas as pl
from jax.experimental.pallas import tpu as pltpu
from jax.experimental.pallas import tpu_sc as plsc

_B = 1024
_D = 32
_VT = 2048
_C = 512
_NW = 32
_SW = _B // _NW


def _q2_body(x_ref, w2_ref, b2_ref, o_ref, oxc_ref):
    xb = x_ref[...]
    q2 = lax.dot_general(w2_ref[...], xb, (((1,), (1,)), ((), ())),
                         preferred_element_type=jnp.float32) + b2_ref[...]
    o_ref[...] = q2[None]
    oxc_ref[...] = xb[None]


def _sc_prep_body(xc_hbm, q2c_hbm, batch_hbm, wts_hbm, sv_hbm, out_hbm,
                  batch_v, wts_v, sv_v, vn_v, q1s_v, sg_v,
                  x_v, q2_v, sh_v):
    n = batch_hbm.shape[0]
    wid = lax.axis_index("s") * 2 + lax.axis_index("c")
    s0 = wid * _SW

    pltpu.sync_copy(batch_hbm, batch_v)
    pltpu.sync_copy(wts_hbm, wts_v)
    pltpu.sync_copy(sv_hbm, sv_v)

    iota16 = lax.iota(jnp.int32, 16)

    def lower_bound(thr):
        pos = jnp.zeros((16,), jnp.int32)
        for j in range(14, -1, -1):
            npos = pos + jnp.int32(1 << j)
            idx = jnp.minimum(npos - 1, n - 1)
            val = plsc.load_gather(batch_v, [idx])
            pred = (npos <= n) & (val < thr)
            pos = jnp.where(pred, npos, pos)
        return pos

    e0 = lower_bound(s0 + 1 + iota16)
    e1 = lower_bound(s0 + 17 + iota16)
    lo = lower_bound(jnp.full((16,), s0, jnp.int32))
    t_lo = lo[0]
    t_hi = e1[15]

    tv0 = jnp.where(e0 - 1 < 0, n - 1, e0 - 1)
    tv1 = jnp.where(e1 - 1 < 0, n - 1, e1 - 1)

    c_lo = t_lo // _C
    c_hi = (t_hi + _C - 1) // _C
    tmin = jnp.minimum(jnp.min(tv0), jnp.min(tv1))
    tmax = jnp.maximum(jnp.max(tv0), jnp.max(tv1))
    a_lo = jnp.minimum(c_lo, tmin // _C)
    a_hi = jnp.maximum(c_hi, tmax // _C + 1)

    def chunk_vn(c, carry):
        base = c * _C
        pltpu.sync_copy(xc_hbm.at[c], x_v)
        for sloc in range(_SW):
            t = tv0[sloc] if sloc < 16 else tv1[sloc - 16]

            @pl.when((t >= base) & (t < base + _C))
            def _copy_row(sloc=sloc, t=t, base=base):
                vn_v[pl.ds(sloc * _D, 16)] = x_v[t - base, pl.ds(0, 16)]
                vn_v[pl.ds(sloc * _D + 16, 16)] = x_v[t - base, pl.ds(16, 16)]

        return carry

    lax.fori_loop(a_lo, a_hi, chunk_vn, 0)

    def q1s_one(sloc, carry):
        acc0 = sv_v[pl.ds(0, 16)]
        acc1 = sv_v[pl.ds(16, 16)]
        v0 = vn_v[pl.ds(sloc * _D, 16)]
        v1 = vn_v[pl.ds(sloc * _D + 16, 16)]
        for k in range(_D):
            sck = v0[k] if k < 16 else v1[k - 16]
            acc0 = acc0 + sck * wts_v[pl.ds(k * _D, 16)]
            acc1 = acc1 + sck * wts_v[pl.ds(k * _D + 16, 16)]
        q1s_v[pl.ds(sloc * _D, 16)] = acc0
        q1s_v[pl.ds(sloc * _D + 16, 16)] = acc1
        return carry

    lax.fori_loop(0, _SW, q1s_one, 0)

    def zero_one(i, carry):
        sg_v[pl.ds(i * 16, 16)] = jnp.zeros((16,), jnp.float32)
        return carry

    lax.fori_loop(0, _SW * _D // 16, zero_one, 0)

    def chunk(c, carry):
        base = c * _C
        pltpu.sync_copy(xc_hbm.at[c], x_v)
        pltpu.sync_copy(q2c_hbm.at[c], q2_v)
        j0 = jnp.maximum(t_lo - base, 0)
        j1 = jnp.minimum(t_hi - base, _C)

        def blk(g, carry2):
            off = g * 16
            tok = off + iota16
            msk = (tok >= j0) & (tok < j1)
            segv = batch_v[pl.ds(base + off, 16)]
            sloc = jnp.clip(segv - s0, 0, _SW - 1)
            acc = sv_v[pl.ds(96, 16)]
            qw0 = sv_v[pl.ds(32, 16)]
            qw1 = sv_v[pl.ds(48, 16)]
            for dd in range(_D):
                q1g = plsc.load_gather(q1s_v, [sloc * _D + dd])
                sgm = 1.0 / (1.0 + jnp.exp(-(q1g + q2_v[dd, pl.ds(off, 16)])))
                qwd = qw0[dd] if dd < 16 else qw1[dd - 16]
                acc = acc + qwd * sgm
            accm = jnp.where(msk, acc, 0.0)
            for j in range(16):
                a = accm[j]
                slj = sloc[j]
                r0 = sg_v[pl.ds(slj * _D, 16)]
                r1 = sg_v[pl.ds(slj * _D + 16, 16)]
                sg_v[pl.ds(slj * _D, 16)] = r0 + a * x_v[off + j, pl.ds(0, 16)]
                sg_v[pl.ds(slj * _D + 16, 16)] = (
                    r1 + a * x_v[off + j, pl.ds(16, 16)])
            return carry2

        lax.fori_loop(0, _C // 16, blk, 0)
        return carry

    lax.fori_loop(c_lo, c_hi, chunk, 0)

    def sh_one(sloc, carry):
        acc0 = sv_v[pl.ds(64, 16)]
        acc1 = sv_v[pl.ds(80, 16)]
        v0 = vn_v[pl.ds(sloc * _D, 16)]
        v1 = vn_v[pl.ds(sloc * _D + 16, 16)]
        g0 = sg_v[pl.ds(sloc * _D, 16)]
        g1 = sg_v[pl.ds(sloc * _D + 16, 16)]
        for k in range(_D):
            vv = v0[k] if k < 16 else v1[k - 16]
            gg = g0[k] if k < 16 else g1[k - 16]
            acc0 = (acc0 + vv * wts_v[pl.ds(1024 + k * _D, 16)]
                    + gg * wts_v[pl.ds(2048 + k * _D, 16)])
            acc1 = (acc1 + vv * wts_v[pl.ds(1024 + k * _D + 16, 16)]
                    + gg * wts_v[pl.ds(2048 + k * _D + 16, 16)])
        sh_v[sloc, pl.ds(0, 16)] = acc0
        sh_v[sloc, pl.ds(16, 16)] = acc1
        return carry

    lax.fori_loop(0, _SW, sh_one, 0)
    pltpu.sync_copy(sh_v, out_hbm.at[pl.ds(s0, _SW)])


_sc_prep = functools.partial(
    pl.kernel,
    _sc_prep_body,
    out_type=jax.ShapeDtypeStruct((_B, _D), jnp.float32),
    mesh=plsc.VectorSubcoreMesh(core_axis_name="c", subcore_axis_name="s"),
    compiler_params=pltpu.CompilerParams(needs_layout_passes=False),
    scratch_types=[
        pltpu.VMEM((16384,), jnp.int32),
        pltpu.VMEM((3 * _D * _D,), jnp.float32),
        pltpu.VMEM((128,), jnp.float32),
        pltpu.VMEM((_SW * _D,), jnp.float32),
        pltpu.VMEM((_SW * _D,), jnp.float32),
        pltpu.VMEM((_SW * _D,), jnp.float32),
        pltpu.VMEM((_C, _D), jnp.float32),
        pltpu.VMEM((_D, _C), jnp.float32),
        pltpu.VMEM((_SW, _D), jnp.float32),
    ],
)


def _score_body(e_ref, sh_ref, out_ref):
    out_ref[...] = lax.dot_general(e_ref[...].astype(jnp.bfloat16),
                                   sh_ref[...].astype(jnp.bfloat16),
                                   (((1,), (1,)), ((), ())),
                                   preferred_element_type=jnp.float32)


def kernel(session_embedding, all_item_embedding, batch,
           W1_w, W1_b, W2_w, W2_b, q_w, q_b, W3_w, W3_b):
    n, d = session_embedding.shape
    v = all_item_embedding.shape[0]

    batch = batch.astype(jnp.int32)

    q2c, xc = pl.pallas_call(
        _q2_body,
        grid=(n // _C,),
        in_specs=[pl.BlockSpec((_C, d), lambda c: (c, 0)),
                  pl.BlockSpec((d, d), lambda c: (0, 0)),
                  pl.BlockSpec((d, 1), lambda c: (0, 0))],
        out_specs=[pl.BlockSpec((1, d, _C), lambda c: (c, 0, 0)),
                   pl.BlockSpec((1, _C, d), lambda c: (c, 0, 0))],
        out_shape=[jax.ShapeDtypeStruct((n // _C, d, _C), jnp.float32),
                   jax.ShapeDtypeStruct((n // _C, _C, d), jnp.float32)],
    )(session_embedding, W2_w, W2_b[:, None])

    wts = jnp.concatenate([W1_w.T, W3_w[:, :d].T, W3_w[:, d:].T],
                          axis=0).reshape(-1)
    sv = jnp.concatenate([W1_b, q_w[0], W3_b,
                          jnp.full((32,), q_b[0], jnp.float32)])

    sh = _sc_prep()(xc, q2c, batch, wts, sv)

    nvt = pl.cdiv(v, _VT)
    zt = pl.pallas_call(
        _score_body,
        grid=(nvt,),
        in_specs=[pl.BlockSpec((_VT, d), lambda i: (i, 0)),
                  pl.BlockSpec((_B, d), lambda i: (0, 0))],
        out_specs=pl.BlockSpec((_VT, _B), lambda i: (i, 0)),
        out_shape=jax.ShapeDtypeStruct((v, _B), jnp.float32),
        compiler_params=pltpu.CompilerParams(
            dimension_semantics=("arbitrary",)),
    )(all_item_embedding, sh)
    return zt.T


# speedup vs baseline: 1.0571x; 1.0474x over previous
import functools

import jax
import jax.numpy as jnp
from jax import lax
from jax.experimental import pallas as pl
from jax.experimental.pallas import tpu as pltpu
from jax.experimental.pallas import tpu_sc as plsc

_B = 1024
_D = 32
_VT = 2048
_C = 512
_NW = 32
_SW = _B // _NW


def _q2_body(x_ref, w2_ref, b2_ref, o_ref, oxc_ref):
    xb = x_ref[...]
    q2 = lax.dot_general(w2_ref[...], xb, (((1,), (1,)), ((), ())),
                         preferred_element_type=jnp.float32) + b2_ref[...]
    o_ref[...] = q2[None]
    oxc_ref[...] = xb[None]


def _sc_prep_body(xc_hbm, q2c_hbm, batch_hbm, wts_hbm, sv_hbm, out_hbm,
                  batch_v, wts_v, sv_v, vn_v, q1s_v, sg_v,
                  x_v, q2_v, sh_v):
    n = batch_hbm.shape[0]
    wid = lax.axis_index("s") * 2 + lax.axis_index("c")
    s0 = wid * _SW

    pltpu.sync_copy(batch_hbm, batch_v)
    pltpu.sync_copy(wts_hbm, wts_v)
    pltpu.sync_copy(sv_hbm, sv_v)

    iota16 = lax.iota(jnp.int32, 16)

    def lower_bound(thr):
        pos = jnp.zeros((16,), jnp.int32)
        for j in range(14, -1, -1):
            npos = pos + jnp.int32(1 << j)
            idx = jnp.minimum(npos - 1, n - 1)
            val = plsc.load_gather(batch_v, [idx])
            pred = (npos <= n) & (val < thr)
            pos = jnp.where(pred, npos, pos)
        return pos

    e0 = lower_bound(s0 + 1 + iota16)
    e1 = lower_bound(s0 + 17 + iota16)
    lo = lower_bound(jnp.full((16,), s0, jnp.int32))
    t_lo = lo[0]
    t_hi = e1[15]

    tv0 = jnp.where(e0 - 1 < 0, n - 1, e0 - 1)
    tv1 = jnp.where(e1 - 1 < 0, n - 1, e1 - 1)

    c_lo = t_lo // _C
    c_hi = (t_hi + _C - 1) // _C
    tmin = jnp.minimum(jnp.min(tv0), jnp.min(tv1))
    tmax = jnp.maximum(jnp.max(tv0), jnp.max(tv1))
    a_lo = jnp.minimum(c_lo, tmin // _C)
    a_hi = jnp.maximum(c_hi, tmax // _C + 1)

    def chunk_vn(c, carry):
        base = c * _C
        pltpu.sync_copy(xc_hbm.at[c], x_v)
        for sloc in range(_SW):
            t = tv0[sloc] if sloc < 16 else tv1[sloc - 16]

            @pl.when((t >= base) & (t < base + _C))
            def _copy_row(sloc=sloc, t=t, base=base):
                vn_v[pl.ds(sloc * _D, 16)] = x_v[t - base, pl.ds(0, 16)]
                vn_v[pl.ds(sloc * _D + 16, 16)] = x_v[t - base, pl.ds(16, 16)]

        return carry

    lax.fori_loop(a_lo, a_hi, chunk_vn, 0)

    def q1s_one(sloc, carry):
        acc0 = sv_v[pl.ds(0, 16)]
        acc1 = sv_v[pl.ds(16, 16)]
        v0 = vn_v[pl.ds(sloc * _D, 16)]
        v1 = vn_v[pl.ds(sloc * _D + 16, 16)]
        for k in range(_D):
            sck = v0[k] if k < 16 else v1[k - 16]
            acc0 = acc0 + sck * wts_v[pl.ds(k * _D, 16)]
            acc1 = acc1 + sck * wts_v[pl.ds(k * _D + 16, 16)]
        q1s_v[pl.ds(sloc * _D, 16)] = acc0
        q1s_v[pl.ds(sloc * _D + 16, 16)] = acc1
        return carry

    lax.fori_loop(0, _SW, q1s_one, 0)

    def zero_one(i, carry):
        sg_v[pl.ds(i * 16, 16)] = jnp.zeros((16,), jnp.float32)
        return carry

    lax.fori_loop(0, _SW * _D // 16, zero_one, 0)

    def chunk(c, carry):
        base = c * _C
        pltpu.sync_copy(xc_hbm.at[c], x_v)
        pltpu.sync_copy(q2c_hbm.at[c], q2_v)
        j0 = jnp.maximum(t_lo - base, 0)
        j1 = jnp.minimum(t_hi - base, _C)

        def blk(g, carry2):
            off = g * 16
            tok = off + iota16
            msk = (tok >= j0) & (tok < j1)
            segv = batch_v[pl.ds(base + off, 16)]
            sloc = jnp.clip(segv - s0, 0, _SW - 1)
            acc = sv_v[pl.ds(96, 16)]
            qw0 = sv_v[pl.ds(32, 16)]
            qw1 = sv_v[pl.ds(48, 16)]
            for dd in range(_D):
                q1g = plsc.load_gather(q1s_v, [sloc * _D + dd])
                sgm = 1.0 / (1.0 + jnp.exp(-(q1g + q2_v[dd, pl.ds(off, 16)])))
                qwd = qw0[dd] if dd < 16 else qw1[dd - 16]
                acc = acc + qwd * sgm
            accm = jnp.where(msk, acc, 0.0)
            for j in range(16):
                a = accm[j]
                slj = sloc[j]
                r0 = sg_v[pl.ds(slj * _D, 16)]
                r1 = sg_v[pl.ds(slj * _D + 16, 16)]
                sg_v[pl.ds(slj * _D, 16)] = r0 + a * x_v[off + j, pl.ds(0, 16)]
                sg_v[pl.ds(slj * _D + 16, 16)] = (
                    r1 + a * x_v[off + j, pl.ds(16, 16)])
            return carry2

        lax.fori_loop(0, _C // 16, blk, 0)
        return carry

    lax.fori_loop(c_lo, c_hi, chunk, 0)

    def sh_one(sloc, carry):
        acc0 = sv_v[pl.ds(64, 16)]
        acc1 = sv_v[pl.ds(80, 16)]
        v0 = vn_v[pl.ds(sloc * _D, 16)]
        v1 = vn_v[pl.ds(sloc * _D + 16, 16)]
        g0 = sg_v[pl.ds(sloc * _D, 16)]
        g1 = sg_v[pl.ds(sloc * _D + 16, 16)]
        for k in range(_D):
            vv = v0[k] if k < 16 else v1[k - 16]
            gg = g0[k] if k < 16 else g1[k - 16]
            acc0 = (acc0 + vv * wts_v[pl.ds(1024 + k * _D, 16)]
                    + gg * wts_v[pl.ds(2048 + k * _D, 16)])
            acc1 = (acc1 + vv * wts_v[pl.ds(1024 + k * _D + 16, 16)]
                    + gg * wts_v[pl.ds(2048 + k * _D + 16, 16)])
        sh_v[sloc, pl.ds(0, 16)] = acc0
        sh_v[sloc, pl.ds(16, 16)] = acc1
        return carry

    lax.fori_loop(0, _SW, sh_one, 0)
    pltpu.sync_copy(sh_v, out_hbm.at[pl.ds(s0, _SW)])


_sc_prep = functools.partial(
    pl.kernel,
    _sc_prep_body,
    out_type=jax.ShapeDtypeStruct((_B, _D), jnp.float32),
    mesh=plsc.VectorSubcoreMesh(core_axis_name="c", subcore_axis_name="s"),
    compiler_params=pltpu.CompilerParams(needs_layout_passes=False),
    scratch_types=[
        pltpu.VMEM((16384,), jnp.int32),
        pltpu.VMEM((3 * _D * _D,), jnp.float32),
        pltpu.VMEM((128,), jnp.float32),
        pltpu.VMEM((_SW * _D,), jnp.float32),
        pltpu.VMEM((_SW * _D,), jnp.float32),
        pltpu.VMEM((_SW * _D,), jnp.float32),
        pltpu.VMEM((_C, _D), jnp.float32),
        pltpu.VMEM((_D, _C), jnp.float32),
        pltpu.VMEM((_SW, _D), jnp.float32),
    ],
)


def _score_body(e_ref, sh_ref, out_ref):
    out_ref[...] = lax.dot_general(e_ref[...], sh_ref[...],
                                   (((1,), (1,)), ((), ())),
                                   preferred_element_type=jnp.float32)


def kernel(session_embedding, all_item_embedding, batch,
           W1_w, W1_b, W2_w, W2_b, q_w, q_b, W3_w, W3_b):
    n, d = session_embedding.shape
    v = all_item_embedding.shape[0]

    batch = batch.astype(jnp.int32)

    q2c, xc = pl.pallas_call(
        _q2_body,
        grid=(n // _C,),
        in_specs=[pl.BlockSpec((_C, d), lambda c: (c, 0)),
                  pl.BlockSpec((d, d), lambda c: (0, 0)),
                  pl.BlockSpec((d, 1), lambda c: (0, 0))],
        out_specs=[pl.BlockSpec((1, d, _C), lambda c: (c, 0, 0)),
                   pl.BlockSpec((1, _C, d), lambda c: (c, 0, 0))],
        out_shape=[jax.ShapeDtypeStruct((n // _C, d, _C), jnp.float32),
                   jax.ShapeDtypeStruct((n // _C, _C, d), jnp.float32)],
    )(session_embedding, W2_w, W2_b[:, None])

    wts = jnp.concatenate([W1_w.T, W3_w[:, :d].T, W3_w[:, d:].T],
                          axis=0).reshape(-1)
    sv = jnp.concatenate([W1_b, q_w[0], W3_b,
                          jnp.full((32,), q_b[0], jnp.float32)])

    sh = _sc_prep()(xc, q2c, batch, wts, sv)

    nvt = pl.cdiv(v, _VT)
    zt = pl.pallas_call(
        _score_body,
        grid=(nvt,),
        in_specs=[pl.BlockSpec((_VT, d), lambda i: (i, 0)),
                  pl.BlockSpec((_B, d), lambda i: (0, 0))],
        out_specs=pl.BlockSpec((_VT, _B), lambda i: (i, 0)),
        out_shape=jax.ShapeDtypeStruct((v, _B), jnp.float32),
        compiler_params=pltpu.CompilerParams(
            dimension_semantics=("arbitrary",)),
    )(all_item_embedding.astype(jnp.bfloat16), sh.astype(jnp.bfloat16))
    return zt.T


# speedup vs baseline: 1.1726x; 1.1093x over previous
import functools

import jax
import jax.numpy as jnp
from jax import lax
from jax.experimental import pallas as pl
from jax.experimental.pallas import tpu as pltpu
from jax.experimental.pallas import tpu_sc as plsc

_B = 1024
_D = 32
_VT = 2048
_C = 512
_NW = 32
_SW = _B // _NW


def _q2_body(x_ref, w2_ref, b2_ref, o_ref, oxc_ref):
    xb = x_ref[...]
    q2 = lax.dot_general(w2_ref[...], xb, (((1,), (1,)), ((), ())),
                         preferred_element_type=jnp.float32) + b2_ref[...]
    o_ref[...] = q2[None]
    oxc_ref[...] = xb[None]


def _sc_prep_body(xc_hbm, q2c_hbm, batch_hbm, wts_hbm, sv_hbm, out_hbm,
                  batch_v, wts_v, sv_v, vn_v, q1s_v, sg_v,
                  x_v, q2_v, sh_v):
    n = batch_hbm.shape[0]
    wid = lax.axis_index("s") * 2 + lax.axis_index("c")
    s0 = wid * _SW

    pltpu.sync_copy(batch_hbm, batch_v)
    pltpu.sync_copy(wts_hbm, wts_v)
    pltpu.sync_copy(sv_hbm, sv_v)

    iota16 = lax.iota(jnp.int32, 16)

    def lower_bound(thr):
        pos = jnp.zeros((16,), jnp.int32)
        for j in range(14, -1, -1):
            npos = pos + jnp.int32(1 << j)
            idx = jnp.minimum(npos - 1, n - 1)
            val = plsc.load_gather(batch_v, [idx])
            pred = (npos <= n) & (val < thr)
            pos = jnp.where(pred, npos, pos)
        return pos

    e0 = lower_bound(s0 + 1 + iota16)
    e1 = lower_bound(s0 + 17 + iota16)
    lo = lower_bound(jnp.full((16,), s0, jnp.int32))
    t_lo = lo[0]
    t_hi = e1[15]

    tv0 = jnp.where(e0 - 1 < 0, n - 1, e0 - 1)
    tv1 = jnp.where(e1 - 1 < 0, n - 1, e1 - 1)

    c_lo = t_lo // _C
    c_hi = (t_hi + _C - 1) // _C
    tmin = jnp.minimum(jnp.min(tv0), jnp.min(tv1))
    tmax = jnp.maximum(jnp.max(tv0), jnp.max(tv1))
    a_lo = jnp.minimum(c_lo, tmin // _C)
    a_hi = jnp.maximum(c_hi, tmax // _C + 1)

    def chunk_vn(c, carry):
        base = c * _C
        pltpu.sync_copy(xc_hbm.at[c], x_v)
        for sloc in range(_SW):
            t = tv0[sloc] if sloc < 16 else tv1[sloc - 16]

            @pl.when((t >= base) & (t < base + _C))
            def _copy_row(sloc=sloc, t=t, base=base):
                vn_v[pl.ds(sloc * _D, 16)] = x_v[t - base, pl.ds(0, 16)]
                vn_v[pl.ds(sloc * _D + 16, 16)] = x_v[t - base, pl.ds(16, 16)]

        return carry

    lax.fori_loop(a_lo, a_hi, chunk_vn, 0)

    def q1s_one(sloc, carry):
        acc0 = sv_v[pl.ds(0, 16)]
        acc1 = sv_v[pl.ds(16, 16)]
        v0 = vn_v[pl.ds(sloc * _D, 16)]
        v1 = vn_v[pl.ds(sloc * _D + 16, 16)]
        for k in range(_D):
            sck = v0[k] if k < 16 else v1[k - 16]
            acc0 = acc0 + sck * wts_v[pl.ds(k * _D, 16)]
            acc1 = acc1 + sck * wts_v[pl.ds(k * _D + 16, 16)]
        q1s_v[pl.ds(sloc * _D, 16)] = acc0
        q1s_v[pl.ds(sloc * _D + 16, 16)] = acc1
        return carry

    lax.fori_loop(0, _SW, q1s_one, 0)

    def zero_one(i, carry):
        sg_v[pl.ds(i * 16, 16)] = jnp.zeros((16,), jnp.float32)
        return carry

    lax.fori_loop(0, _SW * _D // 16, zero_one, 0)

    def chunk(c, carry):
        base = c * _C
        pltpu.sync_copy(xc_hbm.at[c], x_v)
        pltpu.sync_copy(q2c_hbm.at[c], q2_v)
        j0 = jnp.maximum(t_lo - base, 0)
        j1 = jnp.minimum(t_hi - base, _C)

        def blk(g, carry2):
            off = g * 16
            tok = off + iota16
            msk = (tok >= j0) & (tok < j1)
            segv = batch_v[pl.ds(base + off, 16)]
            sloc = jnp.clip(segv - s0, 0, _SW - 1)
            acc = sv_v[pl.ds(96, 16)]
            qw0 = sv_v[pl.ds(32, 16)]
            qw1 = sv_v[pl.ds(48, 16)]
            for dd in range(_D):
                q1g = plsc.load_gather(q1s_v, [sloc * _D + dd])
                sgm = 1.0 / (1.0 + jnp.exp(-(q1g + q2_v[dd, pl.ds(off, 16)])))
                qwd = qw0[dd] if dd < 16 else qw1[dd - 16]
                acc = acc + qwd * sgm
            accm = jnp.where(msk, acc, 0.0)
            for j in range(16):
                a = accm[j]
                slj = sloc[j]
                r0 = sg_v[pl.ds(slj * _D, 16)]
                r1 = sg_v[pl.ds(slj * _D + 16, 16)]
                sg_v[pl.ds(slj * _D, 16)] = r0 + a * x_v[off + j, pl.ds(0, 16)]
                sg_v[pl.ds(slj * _D + 16, 16)] = (
                    r1 + a * x_v[off + j, pl.ds(16, 16)])
            return carry2

        lax.fori_loop(j0 // 16, (j1 + 15) // 16, blk, 0)
        return carry

    lax.fori_loop(c_lo, c_hi, chunk, 0)

    def sh_one(sloc, carry):
        acc0 = sv_v[pl.ds(64, 16)]
        acc1 = sv_v[pl.ds(80, 16)]
        v0 = vn_v[pl.ds(sloc * _D, 16)]
        v1 = vn_v[pl.ds(sloc * _D + 16, 16)]
        g0 = sg_v[pl.ds(sloc * _D, 16)]
        g1 = sg_v[pl.ds(sloc * _D + 16, 16)]
        for k in range(_D):
            vv = v0[k] if k < 16 else v1[k - 16]
            gg = g0[k] if k < 16 else g1[k - 16]
            acc0 = (acc0 + vv * wts_v[pl.ds(1024 + k * _D, 16)]
                    + gg * wts_v[pl.ds(2048 + k * _D, 16)])
            acc1 = (acc1 + vv * wts_v[pl.ds(1024 + k * _D + 16, 16)]
                    + gg * wts_v[pl.ds(2048 + k * _D + 16, 16)])
        sh_v[sloc, pl.ds(0, 16)] = acc0
        sh_v[sloc, pl.ds(16, 16)] = acc1
        return carry

    lax.fori_loop(0, _SW, sh_one, 0)
    pltpu.sync_copy(sh_v, out_hbm.at[pl.ds(s0, _SW)])


_sc_prep = functools.partial(
    pl.kernel,
    _sc_prep_body,
    out_type=jax.ShapeDtypeStruct((_B, _D), jnp.float32),
    mesh=plsc.VectorSubcoreMesh(core_axis_name="c", subcore_axis_name="s"),
    compiler_params=pltpu.CompilerParams(needs_layout_passes=False),
    scratch_types=[
        pltpu.VMEM((16384,), jnp.int32),
        pltpu.VMEM((3 * _D * _D,), jnp.float32),
        pltpu.VMEM((128,), jnp.float32),
        pltpu.VMEM((_SW * _D,), jnp.float32),
        pltpu.VMEM((_SW * _D,), jnp.float32),
        pltpu.VMEM((_SW * _D,), jnp.float32),
        pltpu.VMEM((_C, _D), jnp.float32),
        pltpu.VMEM((_D, _C), jnp.float32),
        pltpu.VMEM((_SW, _D), jnp.float32),
    ],
)


def _score_body(e_ref, sh_ref, out_ref):
    out_ref[...] = lax.dot_general(e_ref[...], sh_ref[...],
                                   (((1,), (1,)), ((), ())),
                                   preferred_element_type=jnp.float32)


def kernel(session_embedding, all_item_embedding, batch,
           W1_w, W1_b, W2_w, W2_b, q_w, q_b, W3_w, W3_b):
    n, d = session_embedding.shape
    v = all_item_embedding.shape[0]

    batch = batch.astype(jnp.int32)

    q2c, xc = pl.pallas_call(
        _q2_body,
        grid=(n // _C,),
        in_specs=[pl.BlockSpec((_C, d), lambda c: (c, 0)),
                  pl.BlockSpec((d, d), lambda c: (0, 0)),
                  pl.BlockSpec((d, 1), lambda c: (0, 0))],
        out_specs=[pl.BlockSpec((1, d, _C), lambda c: (c, 0, 0)),
                   pl.BlockSpec((1, _C, d), lambda c: (c, 0, 0))],
        out_shape=[jax.ShapeDtypeStruct((n // _C, d, _C), jnp.float32),
                   jax.ShapeDtypeStruct((n // _C, _C, d), jnp.float32)],
    )(session_embedding, W2_w, W2_b[:, None])

    wts = jnp.concatenate([W1_w.T, W3_w[:, :d].T, W3_w[:, d:].T],
                          axis=0).reshape(-1)
    sv = jnp.concatenate([W1_b, q_w[0], W3_b,
                          jnp.full((32,), q_b[0], jnp.float32)])

    sh = _sc_prep()(xc, q2c, batch, wts, sv)

    nvt = pl.cdiv(v, _VT)
    zt = pl.pallas_call(
        _score_body,
        grid=(nvt,),
        in_specs=[pl.BlockSpec((_VT, d), lambda i: (i, 0)),
                  pl.BlockSpec((_B, d), lambda i: (0, 0))],
        out_specs=pl.BlockSpec((_VT, _B), lambda i: (i, 0)),
        out_shape=jax.ShapeDtypeStruct((v, _B), jnp.float32),
        compiler_params=pltpu.CompilerParams(
            dimension_semantics=("arbitrary",)),
    )(all_item_embedding.astype(jnp.bfloat16), sh.astype(jnp.bfloat16))
    return zt.T


# speedup vs baseline: 1.1792x; 1.0056x over previous
import jax
import jax.numpy as jnp
from jax import lax
from jax.experimental import pallas as pl
from jax.experimental.pallas import tpu as pltpu
from jax.experimental.pallas import tpu_sc as plsc

_B = 1024
_D = 32
_VT = 2048
_C = 512
_NW = 32
_SW = _B // _NW


def _q2_body(x_ref, w2_ref, b2_ref, o_ref, oxc_ref):
    xb = x_ref[...]
    q2 = lax.dot_general(w2_ref[...], xb, (((1,), (1,)), ((), ())),
                         preferred_element_type=jnp.float32) + b2_ref[...]
    o_ref[...] = q2[None]
    oxc_ref[...] = xb[None]


def _sc_prep_body(xc_hbm, q2c_hbm, batch_hbm, wts_hbm, sv_hbm, out_hbm,
                  batch_v, wts_v, sv_v, vn_v, q1s_v, sg_v,
                  x_v, q2_v, sh_v):
    n = batch_hbm.shape[0]
    wid = lax.axis_index("s") * 2 + lax.axis_index("c")
    s0 = wid * _SW

    pltpu.sync_copy(batch_hbm, batch_v)
    pltpu.sync_copy(wts_hbm, wts_v)
    pltpu.sync_copy(sv_hbm, sv_v)

    iota16 = lax.iota(jnp.int32, 16)

    def lower_bound(thr):
        pos = jnp.zeros((16,), jnp.int32)
        for j in range(14, -1, -1):
            npos = pos + jnp.int32(1 << j)
            idx = jnp.minimum(npos - 1, n - 1)
            val = plsc.load_gather(batch_v, [idx])
            pred = (npos <= n) & (val < thr)
            pos = jnp.where(pred, npos, pos)
        return pos

    e0 = lower_bound(s0 + 1 + iota16)
    e1 = lower_bound(s0 + 17 + iota16)
    lo = lower_bound(jnp.full((16,), s0, jnp.int32))
    t_lo = lo[0]
    t_hi = e1[15]

    tv0 = jnp.where(e0 - 1 < 0, n - 1, e0 - 1)
    tv1 = jnp.where(e1 - 1 < 0, n - 1, e1 - 1)

    c_lo = t_lo // _C
    c_hi = (t_hi + _C - 1) // _C
    tmin = jnp.minimum(jnp.min(tv0), jnp.min(tv1))
    tmax = jnp.maximum(jnp.max(tv0), jnp.max(tv1))
    a_lo = jnp.minimum(c_lo, tmin // _C)
    a_hi = jnp.maximum(c_hi, tmax // _C + 1)

    def chunk_vn(c, carry):
        base = c * _C
        pltpu.sync_copy(xc_hbm.at[c], x_v)
        for sloc in range(_SW):
            t = tv0[sloc] if sloc < 16 else tv1[sloc - 16]

            @pl.when((t >= base) & (t < base + _C))
            def _copy_row(sloc=sloc, t=t, base=base):
                vn_v[pl.ds(sloc * _D, 16)] = x_v[t - base, pl.ds(0, 16)]
                vn_v[pl.ds(sloc * _D + 16, 16)] = x_v[t - base, pl.ds(16, 16)]

        return carry

    lax.fori_loop(a_lo, a_hi, chunk_vn, 0)

    def q1s_one(sloc, carry):
        acc0 = sv_v[pl.ds(0, 16)]
        acc1 = sv_v[pl.ds(16, 16)]
        v0 = vn_v[pl.ds(sloc * _D, 16)]
        v1 = vn_v[pl.ds(sloc * _D + 16, 16)]
        for k in range(_D):
            sck = v0[k] if k < 16 else v1[k - 16]
            acc0 = acc0 + sck * wts_v[pl.ds(k * _D, 16)]
            acc1 = acc1 + sck * wts_v[pl.ds(k * _D + 16, 16)]
        q1s_v[pl.ds(sloc * _D, 16)] = acc0
        q1s_v[pl.ds(sloc * _D + 16, 16)] = acc1
        return carry

    lax.fori_loop(0, _SW, q1s_one, 0)

    def zero_one(i, carry):
        sg_v[pl.ds(i * 16, 16)] = jnp.zeros((16,), jnp.float32)
        return carry

    lax.fori_loop(0, _SW * _D // 16, zero_one, 0)

    def chunk(c, carry):
        base = c * _C
        pltpu.sync_copy(xc_hbm.at[c], x_v)
        pltpu.sync_copy(q2c_hbm.at[c], q2_v)
        j0 = jnp.maximum(t_lo - base, 0)
        j1 = jnp.minimum(t_hi - base, _C)

        def blk(g, carry2):
            off = g * 16
            tok = off + iota16
            msk = (tok >= j0) & (tok < j1)
            segv = batch_v[pl.ds(base + off, 16)]
            sloc = jnp.clip(segv - s0, 0, _SW - 1)
            acc = sv_v[pl.ds(96, 16)]
            qw0 = sv_v[pl.ds(32, 16)]
            qw1 = sv_v[pl.ds(48, 16)]
            for dd in range(_D):
                q1g = plsc.load_gather(q1s_v, [sloc * _D + dd])
                sgm = 1.0 / (1.0 + jnp.exp(-(q1g + q2_v[dd, pl.ds(off, 16)])))
                qwd = qw0[dd] if dd < 16 else qw1[dd - 16]
                acc = acc + qwd * sgm
            accm = jnp.where(msk, acc, 0.0)
            for j in range(16):
                a = accm[j]
                slj = sloc[j]
                r0 = sg_v[pl.ds(slj * _D, 16)]
                r1 = sg_v[pl.ds(slj * _D + 16, 16)]
                sg_v[pl.ds(slj * _D, 16)] = r0 + a * x_v[off + j, pl.ds(0, 16)]
                sg_v[pl.ds(slj * _D + 16, 16)] = (
                    r1 + a * x_v[off + j, pl.ds(16, 16)])
            return carry2

        lax.fori_loop(j0 // 16, (j1 + 15) // 16, blk, 0)
        return carry

    lax.fori_loop(c_lo, c_hi, chunk, 0)

    def sh_one(sloc, carry):
        acc0 = sv_v[pl.ds(64, 16)]
        acc1 = sv_v[pl.ds(80, 16)]
        v0 = vn_v[pl.ds(sloc * _D, 16)]
        v1 = vn_v[pl.ds(sloc * _D + 16, 16)]
        g0 = sg_v[pl.ds(sloc * _D, 16)]
        g1 = sg_v[pl.ds(sloc * _D + 16, 16)]
        for k in range(_D):
            vv = v0[k] if k < 16 else v1[k - 16]
            gg = g0[k] if k < 16 else g1[k - 16]
            acc0 = (acc0 + vv * wts_v[pl.ds(1024 + k * _D, 16)]
                    + gg * wts_v[pl.ds(2048 + k * _D, 16)])
            acc1 = (acc1 + vv * wts_v[pl.ds(1024 + k * _D + 16, 16)]
                    + gg * wts_v[pl.ds(2048 + k * _D + 16, 16)])
        sh_v[sloc, pl.ds(0, 16)] = acc0
        sh_v[sloc, pl.ds(16, 16)] = acc1
        return carry

    lax.fori_loop(0, _SW, sh_one, 0)
    pltpu.sync_copy(sh_v, out_hbm.at[pl.ds(s0, _SW)])


def _sc_prep():
    return pl.kernel(
        _sc_prep_body,
        out_type=jax.ShapeDtypeStruct((_B, _D), jnp.float32),
        mesh=plsc.VectorSubcoreMesh(core_axis_name="c", subcore_axis_name="s"),
        compiler_params=pltpu.CompilerParams(needs_layout_passes=False),
        scratch_types=[
            pltpu.VMEM((16384,), jnp.int32),
            pltpu.VMEM((3 * _D * _D,), jnp.float32),
            pltpu.VMEM((128,), jnp.float32),
            pltpu.VMEM((_SW * _D,), jnp.float32),
            pltpu.VMEM((_SW * _D,), jnp.float32),
            pltpu.VMEM((_SW * _D,), jnp.float32),
            pltpu.VMEM((_C, _D), jnp.float32),
            pltpu.VMEM((_D, _C), jnp.float32),
            pltpu.VMEM((_SW, _D), jnp.float32),
        ],
    )


def _score_body(e_ref, sh_ref, out_ref):
    out_ref[...] = lax.dot_general(e_ref[...], sh_ref[...],
                                   (((1,), (1,)), ((), ())),
                                   preferred_element_type=jnp.float32)


def kernel(session_embedding, all_item_embedding, batch,
           W1_w, W1_b, W2_w, W2_b, q_w, q_b, W3_w, W3_b):
    n, d = session_embedding.shape
    v = all_item_embedding.shape[0]

    batch = batch.astype(jnp.int32)

    q2c, xc = pl.pallas_call(
        _q2_body,
        grid=(n // _C,),
        in_specs=[pl.BlockSpec((_C, d), lambda c: (c, 0)),
                  pl.BlockSpec((d, d), lambda c: (0, 0)),
                  pl.BlockSpec((d, 1), lambda c: (0, 0))],
        out_specs=[pl.BlockSpec((1, d, _C), lambda c: (c, 0, 0)),
                   pl.BlockSpec((1, _C, d), lambda c: (c, 0, 0))],
        out_shape=[jax.ShapeDtypeStruct((n // _C, d, _C), jnp.float32),
                   jax.ShapeDtypeStruct((n // _C, _C, d), jnp.float32)],
    )(session_embedding, W2_w, W2_b[:, None])

    wts = jnp.concatenate([W1_w.T, W3_w[:, :d].T, W3_w[:, d:].T],
                          axis=0).reshape(-1)
    sv = jnp.concatenate([W1_b, q_w[0], W3_b,
                          jnp.full((32,), q_b[0], jnp.float32)])

    sh = _sc_prep()(xc, q2c, batch, wts, sv)

    nvt = pl.cdiv(v, _VT)
    zt = pl.pallas_call(
        _score_body,
        grid=(nvt,),
        in_specs=[pl.BlockSpec((_VT, d), lambda i: (i, 0)),
                  pl.BlockSpec((_B, d), lambda i: (0, 0))],
        out_specs=pl.BlockSpec((_VT, _B), lambda i: (i, 0)),
        out_shape=jax.ShapeDtypeStruct((v, _B), jnp.float32),
        compiler_params=pltpu.CompilerParams(
            dimension_semantics=("arbitrary",)),
    )(all_item_embedding.astype(jnp.bfloat16), sh.astype(jnp.bfloat16))
    return zt.T
